# baseline (speedup 1.0000x reference)
import jax
import jax.numpy as jnp
from jax import lax
from jax.experimental import pallas as pl
from jax.experimental.pallas import tpu as pltpu
from jax.experimental.pallas import tpu_sc as plsc

N = 10000
E = 320000
C = 128
DE = 16
NEG_SLOPE = 0.2

NC = 2
NS = 16
NW = NC * NS
EW = E // NW

K = 40
NCH = EW // K
NG = (K + 15) // 16

NPAD = 10240
RPT = NPAD // NS

_SC_PARAMS = pltpu.CompilerParams(needs_layout_passes=False,
                                  use_tc_tiling_on_sc=False)


def _dense_body(x_ref, wl_ref, wr_ref, bl_ref, br_ref, xl_ref, xr_ref):
    xv = x_ref[...]
    xl_ref[...] = jnp.dot(xv, wl_ref[...],
                          preferred_element_type=jnp.float32) + bl_ref[...]
    xr_ref[...] = jnp.dot(xv, wr_ref[...],
                          preferred_element_type=jnp.float32) + br_ref[...]


def _edge_body(ea_ref, we_ref, ef_ref):
    ef_ref[...] = jnp.dot(ea_ref[...], we_ref[...],
                          preferred_element_type=jnp.float32)


def _final_body(p_ref, pd_ref, bias_ref, out_ref):
    num = p_ref[0] + p_ref[1]
    den = pd_ref[0] + pd_ref[1]
    out_ref[...] = num / (den + 1e-16) + bias_ref[...]


def _sc_body(xl_hbm, xr_hbm, ef_hbm, src3_hbm, dst3_hbm, att_hbm,
             part_hbm, partd_hbm,
             acc_sh, accd_sh,
             src_i, dst_i, sc_i, xl_rows, xr_rows, ef_rows,
             ex_b, accbuf, att_v, zbuf,
             s_ix0, s_ix1, s_xl0, s_xl1, s_xr0, s_xr1, s_ef0, s_ef1,
             s_sc0, s_sc1, s_sd0, s_sd1):
    cid = lax.axis_index("c")
    sid = lax.axis_index("s")
    wid = cid * NS + sid
    zero16 = jnp.zeros((16,), jnp.float32)
    iota = lax.iota(jnp.int32, 16)
    s_ix = [s_ix0, s_ix1]
    s_xl = [s_xl0, s_xl1]
    s_xr = [s_xr0, s_xr1]
    s_ef = [s_ef0, s_ef1]
    s_sc = [s_sc0, s_sc1]
    s_sd = [s_sd0, s_sd1]

    def _z_row(e, _):
        for j in range(C // 16):
            xl_rows[0, e, pl.ds(16 * j, 16)] = zero16
        return 0
    lax.fori_loop(0, K, _z_row, 0)

    def _z1(e, _):
        zbuf[pl.ds(e * 16, 16)] = zero16
        return 0
    lax.fori_loop(0, RPT // 16, _z1, 0)
    row0 = sid * RPT
    for k in range(RPT // K):
        pltpu.sync_copy(xl_rows.at[0], acc_sh.at[pl.ds(row0 + k * K, K)])
    pltpu.sync_copy(zbuf, accd_sh.at[pl.ds(row0, RPT)])
    plsc.subcore_barrier()

    pltpu.sync_copy(att_hbm, att_v)
    att_j = [att_v[pl.ds(16 * j, 16)] for j in range(C // 16)]

    def issue_idx(c, b):
        pltpu.async_copy(src3_hbm.at[wid, c], src_i.at[b, pl.ds(0, K)],
                         s_ix[b])
        pltpu.async_copy(dst3_hbm.at[wid, c], dst_i.at[b, pl.ds(0, K)],
                         s_ix[b])

    def wait_idx(b):
        pltpu.make_async_copy(src3_hbm.at[0, 0], src_i.at[b, pl.ds(0, K)],
                              s_ix[b]).wait()
        pltpu.make_async_copy(src3_hbm.at[0, 0], dst_i.at[b, pl.ds(0, K)],
                              s_ix[b]).wait()

    def issue_gathers(c, b):
        pltpu.async_copy(xl_hbm.at[src_i.at[b, pl.ds(0, K)]],
                         xl_rows.at[b], s_xl[b])
        pltpu.async_copy(xr_hbm.at[dst_i.at[b, pl.ds(0, K)]],
                         xr_rows.at[b], s_xr[b])
        pltpu.async_copy(ef_hbm.at[pl.ds(wid * EW + c * K, K)],
                         ef_rows.at[b], s_ef[b])

    def wait_gathers(b):
        pltpu.make_async_copy(xl_hbm.at[pl.ds(0, K)],
                              xl_rows.at[b], s_xl[b]).wait()
        pltpu.make_async_copy(xl_hbm.at[pl.ds(0, K)],
                              xr_rows.at[b], s_xr[b]).wait()
        pltpu.make_async_copy(ef_hbm.at[pl.ds(0, K)],
                              ef_rows.at[b], s_ef[b]).wait()

    def drain_scatters(b):
        pltpu.make_async_copy(xl_hbm.at[pl.ds(0, K)],
                              xl_rows.at[b], s_sc[b]).wait()
        pltpu.make_async_copy(partd_hbm.at[0, pl.ds(0, K)],
                              ex_b.at[b, pl.ds(0, K)], s_sd[b]).wait()

    def slot(c, b):
        o = b ^ 1

        @pl.when(c >= 1)
        def _():
            drain_scatters(o)

        @pl.when(c + 1 < NCH)
        def _():
            wait_idx(o)
            issue_gathers(c + 1, o)
        wait_gathers(b)

        for off in (0, 16, K - 16):
            sc_i[b, pl.ds(off, 16)] = dst_i[b, pl.ds(off, 16)]

        def _edge_acc(e, _):
            acc0 = zero16
            acc1 = zero16
            for j in range(C // 16):
                m = (xl_rows[b, e, pl.ds(16 * j, 16)]
                     + xr_rows[b, e, pl.ds(16 * j, 16)]
                     + ef_rows[b, e, pl.ds(16 * j, 16)])
                m = jnp.maximum(m, NEG_SLOPE * m)
                if j % 2 == 0:
                    acc0 = acc0 + m * att_j[j]
                else:
                    acc1 = acc1 + m * att_j[j]
            accbuf[pl.ds(e * 16, 16)] = acc0 + acc1
            return 0
        lax.fori_loop(0, K, _edge_acc, 0, unroll=4)

        for g in range(NG):
            al = [zero16] * 4
            for col in range(16):
                al[col % 4] = al[col % 4] + plsc.load_gather(
                    accbuf, [iota * 16 + (col + g * 256)])
            ex_b[b, pl.ds(g * 16, 16)] = jnp.exp((al[0] + al[1])
                                                 + (al[2] + al[3]))

        def _edge_scale(e, _):
            s = plsc.load_gather(ex_b.at[b],
                                 [jnp.zeros((16,), jnp.int32) + e])
            for j in range(C // 16):
                xl_rows[b, e, pl.ds(16 * j, 16)] = (
                    xl_rows[b, e, pl.ds(16 * j, 16)] * s)
            return 0
        lax.fori_loop(0, K, _edge_scale, 0, unroll=4)

        pltpu.async_copy(xl_rows.at[b], acc_sh.at[sc_i.at[b]], s_sc[b],
                         add=True)
        pltpu.async_copy(ex_b.at[b, pl.ds(0, K)], accd_sh.at[sc_i.at[b]],
                         s_sd[b], add=True)

        @pl.when(c + 2 < NCH)
        def _():
            issue_idx(c + 2, b)

    pltpu.sync_copy(src3_hbm.at[wid, 0], src_i.at[0, pl.ds(0, K)])
    pltpu.sync_copy(dst3_hbm.at[wid, 0], dst_i.at[0, pl.ds(0, K)])
    issue_gathers(0, 0)
    issue_idx(1, 1)

    def _pair(p, _):
        slot(2 * p, 0)
        slot(2 * p + 1, 1)
        return 0
    lax.fori_loop(0, NCH // 2, _pair, 0)

    drain_scatters(1)

    plsc.subcore_barrier()
    pltpu.sync_copy(acc_sh.at[pl.ds(row0, RPT)],
                    part_hbm.at[cid, pl.ds(row0, RPT)])
    pltpu.sync_copy(accd_sh.at[pl.ds(row0, RPT)],
                    partd_hbm.at[cid, pl.ds(row0, RPT)])


def _sc_pass(xl, xr, ef, src3, dst3, att):
    mesh = plsc.VectorSubcoreMesh(core_axis_name="c", subcore_axis_name="s",
                                  num_cores=NC, num_subcores=NS)
    f = pl.kernel(
        _sc_body,
        out_type=[jax.ShapeDtypeStruct((NC, NPAD, C), jnp.float32),
                  jax.ShapeDtypeStruct((NC, NPAD), jnp.float32)],
        mesh=mesh,
        scratch_types=[
            pltpu.VMEM_SHARED((NPAD, C), jnp.float32),
            pltpu.VMEM_SHARED((NPAD,), jnp.float32),
            pltpu.VMEM((2, 48), jnp.int32),
            pltpu.VMEM((2, 48), jnp.int32),
            pltpu.VMEM((2, K), jnp.int32),
            pltpu.VMEM((2, K, C), jnp.float32),
            pltpu.VMEM((2, K, C), jnp.float32),
            pltpu.VMEM((2, K, C), jnp.float32),
            pltpu.VMEM((2, NG * 16), jnp.float32),
            pltpu.VMEM((NG * 256, ), jnp.float32),
            pltpu.VMEM((C,), jnp.float32),
            pltpu.VMEM((RPT,), jnp.float32),
        ] + [pltpu.SemaphoreType.DMA] * 12,
        compiler_params=_SC_PARAMS,
    )
    return f(xl, xr, ef, src3, dst3, att)


def kernel(x, edge_index, edge_attr, W_l, b_l, W_r, b_r, W_e, att, bias):
    src = edge_index[0]
    dst = edge_index[1]

    xl, xr = pl.pallas_call(
        _dense_body,
        out_shape=[jax.ShapeDtypeStruct((N, C), jnp.float32),
                   jax.ShapeDtypeStruct((N, C), jnp.float32)],
        grid=(5,),
        in_specs=[pl.BlockSpec((N // 5, C), lambda i: (i, 0)),
                  pl.BlockSpec((C, C), lambda i: (0, 0)),
                  pl.BlockSpec((C, C), lambda i: (0, 0)),
                  pl.BlockSpec((1, C), lambda i: (0, 0)),
                  pl.BlockSpec((1, C), lambda i: (0, 0))],
        out_specs=[pl.BlockSpec((N // 5, C), lambda i: (i, 0)),
                   pl.BlockSpec((N // 5, C), lambda i: (i, 0))],
    )(x, W_l, W_r, b_l.reshape(1, C), b_r.reshape(1, C))

    EB = 8000
    ef = pl.pallas_call(
        _edge_body,
        out_shape=jax.ShapeDtypeStruct((E, C), jnp.float32),
        grid=(E // EB,),
        in_specs=[pl.BlockSpec((EB, DE), lambda i: (i, 0)),
                  pl.BlockSpec((DE, C), lambda i: (0, 0))],
        out_specs=pl.BlockSpec((EB, C), lambda i: (i, 0)),
    )(edge_attr, W_e)

    part, partd = _sc_pass(xl, xr, ef,
                           src.reshape(NW, NCH, K), dst.reshape(NW, NCH, K),
                           att)

    FB = 1024
    out_full = pl.pallas_call(
        _final_body,
        out_shape=jax.ShapeDtypeStruct((NPAD, C), jnp.float32),
        grid=(NPAD // FB,),
        in_specs=[pl.BlockSpec((NC, FB, C), lambda i: (0, i, 0)),
                  pl.BlockSpec((NC, FB, 1), lambda i: (0, i, 0)),
                  pl.BlockSpec((1, C), lambda i: (0, 0))],
        out_specs=pl.BlockSpec((FB, C), lambda i: (i, 0)),
    )(part, partd.reshape(NC, NPAD, 1), bias.reshape(1, C))
    return out_full[:N]


# speedup vs baseline: 1.2227x; 1.2227x over previous
import jax
import jax.numpy as jnp
from jax import lax
from jax.experimental import pallas as pl
from jax.experimental.pallas import tpu as pltpu
from jax.experimental.pallas import tpu_sc as plsc

N = 10000
E = 320000
C = 128
DE = 16
NEG_SLOPE = 0.2

NC = 2
NS = 16
NW = NC * NS
EW = E // NW

K = 40
NCH = EW // K
NG = (K + 15) // 16

NPAD = 10240
RPT = NPAD // NS

_SC_PARAMS = pltpu.CompilerParams(needs_layout_passes=False,
                                  use_tc_tiling_on_sc=False)


def _dense_body(x_ref, wl_ref, wr_ref, bl_ref, br_ref, xl_ref, xr_ref):
    xv = x_ref[...]
    xl_ref[...] = jnp.dot(xv, wl_ref[...],
                          preferred_element_type=jnp.float32) + bl_ref[...]
    xr_ref[...] = jnp.dot(xv, wr_ref[...],
                          preferred_element_type=jnp.float32) + br_ref[...]


def _edge_body(ea_ref, we_ref, ef_ref):
    ef_ref[...] = jnp.dot(ea_ref[...], we_ref[...],
                          preferred_element_type=jnp.float32)


def _final_body(p_ref, pd_ref, bias_ref, out_ref):
    num = p_ref[0] + p_ref[1]
    den = pd_ref[0] + pd_ref[1]
    out_ref[...] = num / (den + 1e-16) + bias_ref[...]


def _sc_body(xl_hbm, xr_hbm, ef_hbm, src3_hbm, dst3_hbm, att_hbm,
             part_hbm, partd_hbm,
             acc_sh, accd_sh,
             src_i, dst_i, sc_i, xl_rows, xr_rows, ef_rows,
             ex_b, accbuf, att_v, zbuf,
             s_ix0, s_ix1, s_xl0, s_xl1, s_xr0, s_xr1, s_ef0, s_ef1,
             s_sc0, s_sc1, s_sd0, s_sd1):
    cid = lax.axis_index("c")
    sid = lax.axis_index("s")
    wid = cid * NS + sid
    zero16 = jnp.zeros((16,), jnp.float32)
    iota = lax.iota(jnp.int32, 16)
    s_ix = [s_ix0, s_ix1]
    s_xl = [s_xl0, s_xl1]
    s_xr = [s_xr0, s_xr1]
    s_ef = [s_ef0, s_ef1]
    s_sc = [s_sc0, s_sc1]
    s_sd = [s_sd0, s_sd1]

    def _z_row(e, _):
        for j in range(C // 16):
            xl_rows[0, e, pl.ds(16 * j, 16)] = zero16
        return 0
    lax.fori_loop(0, K, _z_row, 0)

    def _z1(e, _):
        zbuf[pl.ds(e * 16, 16)] = zero16
        return 0
    lax.fori_loop(0, RPT // 16, _z1, 0)
    row0 = sid * RPT
    for k in range(RPT // K):
        pltpu.sync_copy(xl_rows.at[0], acc_sh.at[pl.ds(row0 + k * K, K)])
    pltpu.sync_copy(zbuf, accd_sh.at[pl.ds(row0, RPT)])
    plsc.subcore_barrier()

    pltpu.sync_copy(att_hbm, att_v)
    att_j = [att_v[pl.ds(16 * j, 16)] for j in range(C // 16)]

    def issue_idx(c, b):
        pltpu.async_copy(src3_hbm.at[wid, c], src_i.at[b, pl.ds(0, K)],
                         s_ix[b])
        pltpu.async_copy(dst3_hbm.at[wid, c], dst_i.at[b, pl.ds(0, K)],
                         s_ix[b])

    def wait_idx(b):
        pltpu.make_async_copy(src3_hbm.at[0, 0], src_i.at[b, pl.ds(0, K)],
                              s_ix[b]).wait()
        pltpu.make_async_copy(src3_hbm.at[0, 0], dst_i.at[b, pl.ds(0, K)],
                              s_ix[b]).wait()

    def issue_gathers(c, b):
        pltpu.async_copy(xl_hbm.at[src_i.at[b, pl.ds(0, K)]],
                         xl_rows.at[b], s_xl[b])
        pltpu.async_copy(xr_hbm.at[dst_i.at[b, pl.ds(0, K)]],
                         xr_rows.at[b], s_xr[b])
        pltpu.async_copy(ef_hbm.at[pl.ds(wid * EW + c * K, K)],
                         ef_rows.at[b], s_ef[b])

    def wait_gathers(b):
        pltpu.make_async_copy(xl_hbm.at[pl.ds(0, K)],
                              xl_rows.at[b], s_xl[b]).wait()
        pltpu.make_async_copy(xl_hbm.at[pl.ds(0, K)],
                              xr_rows.at[b], s_xr[b]).wait()
        pltpu.make_async_copy(ef_hbm.at[pl.ds(0, K)],
                              ef_rows.at[b], s_ef[b]).wait()

    def drain_scatters(b):
        pltpu.make_async_copy(xl_hbm.at[pl.ds(0, K)],
                              xl_rows.at[b], s_sc[b]).wait()
        pltpu.make_async_copy(partd_hbm.at[0, pl.ds(0, K)],
                              ex_b.at[b, pl.ds(0, K)], s_sd[b]).wait()

    def slot(c, b):
        o = b ^ 1

        @pl.when(c >= 1)
        def _():
            drain_scatters(o)

        @pl.when(c + 1 < NCH)
        def _():
            wait_idx(o)
            issue_gathers(c + 1, o)
        wait_gathers(b)

        for off in (0, 16, K - 16):
            sc_i[b, pl.ds(off, 16)] = dst_i[b, pl.ds(off, 16)]

        @plsc.parallel_loop(0, K, unroll=4)
        def _edge_acc(e):
            acc0 = zero16
            acc1 = zero16
            for j in range(C // 16):
                m = (xl_rows[b, e, pl.ds(16 * j, 16)]
                     + xr_rows[b, e, pl.ds(16 * j, 16)]
                     + ef_rows[b, e, pl.ds(16 * j, 16)])
                m = jnp.maximum(m, NEG_SLOPE * m)
                if j % 2 == 0:
                    acc0 = acc0 + m * att_j[j]
                else:
                    acc1 = acc1 + m * att_j[j]
            accbuf[pl.ds(e * 16, 16)] = acc0 + acc1

        for g in range(NG):
            al = [zero16] * 4
            for col in range(16):
                al[col % 4] = al[col % 4] + plsc.load_gather(
                    accbuf, [iota * 16 + (col + g * 256)])
            ex_b[b, pl.ds(g * 16, 16)] = jnp.exp((al[0] + al[1])
                                                 + (al[2] + al[3]))

        @plsc.parallel_loop(0, K, unroll=4)
        def _edge_scale(e):
            s = plsc.load_gather(ex_b.at[b],
                                 [jnp.zeros((16,), jnp.int32) + e])
            for j in range(C // 16):
                xl_rows[b, e, pl.ds(16 * j, 16)] = (
                    xl_rows[b, e, pl.ds(16 * j, 16)] * s)

        pltpu.async_copy(xl_rows.at[b], acc_sh.at[sc_i.at[b]], s_sc[b],
                         add=True)
        pltpu.async_copy(ex_b.at[b, pl.ds(0, K)], accd_sh.at[sc_i.at[b]],
                         s_sd[b], add=True)

        @pl.when(c + 2 < NCH)
        def _():
            issue_idx(c + 2, b)

    pltpu.sync_copy(src3_hbm.at[wid, 0], src_i.at[0, pl.ds(0, K)])
    pltpu.sync_copy(dst3_hbm.at[wid, 0], dst_i.at[0, pl.ds(0, K)])
    issue_gathers(0, 0)
    issue_idx(1, 1)

    def _pair(p, _):
        slot(2 * p, 0)
        slot(2 * p + 1, 1)
        return 0
    lax.fori_loop(0, NCH // 2, _pair, 0)

    drain_scatters(1)

    plsc.subcore_barrier()
    pltpu.sync_copy(acc_sh.at[pl.ds(row0, RPT)],
                    part_hbm.at[cid, pl.ds(row0, RPT)])
    pltpu.sync_copy(accd_sh.at[pl.ds(row0, RPT)],
                    partd_hbm.at[cid, pl.ds(row0, RPT)])


def _sc_pass(xl, xr, ef, src3, dst3, att):
    mesh = plsc.VectorSubcoreMesh(core_axis_name="c", subcore_axis_name="s",
                                  num_cores=NC, num_subcores=NS)
    f = pl.kernel(
        _sc_body,
        out_type=[jax.ShapeDtypeStruct((NC, NPAD, C), jnp.float32),
                  jax.ShapeDtypeStruct((NC, NPAD), jnp.float32)],
        mesh=mesh,
        scratch_types=[
            pltpu.VMEM_SHARED((NPAD, C), jnp.float32),
            pltpu.VMEM_SHARED((NPAD,), jnp.float32),
            pltpu.VMEM((2, 48), jnp.int32),
            pltpu.VMEM((2, 48), jnp.int32),
            pltpu.VMEM((2, K), jnp.int32),
            pltpu.VMEM((2, K, C), jnp.float32),
            pltpu.VMEM((2, K, C), jnp.float32),
            pltpu.VMEM((2, K, C), jnp.float32),
            pltpu.VMEM((2, NG * 16), jnp.float32),
            pltpu.VMEM((NG * 256, ), jnp.float32),
            pltpu.VMEM((C,), jnp.float32),
            pltpu.VMEM((RPT,), jnp.float32),
        ] + [pltpu.SemaphoreType.DMA] * 12,
        compiler_params=_SC_PARAMS,
    )
    return f(xl, xr, ef, src3, dst3, att)


def kernel(x, edge_index, edge_attr, W_l, b_l, W_r, b_r, W_e, att, bias):
    src = edge_index[0]
    dst = edge_index[1]

    xl, xr = pl.pallas_call(
        _dense_body,
        out_shape=[jax.ShapeDtypeStruct((N, C), jnp.float32),
                   jax.ShapeDtypeStruct((N, C), jnp.float32)],
        grid=(5,),
        in_specs=[pl.BlockSpec((N // 5, C), lambda i: (i, 0)),
                  pl.BlockSpec((C, C), lambda i: (0, 0)),
                  pl.BlockSpec((C, C), lambda i: (0, 0)),
                  pl.BlockSpec((1, C), lambda i: (0, 0)),
                  pl.BlockSpec((1, C), lambda i: (0, 0))],
        out_specs=[pl.BlockSpec((N // 5, C), lambda i: (i, 0)),
                   pl.BlockSpec((N // 5, C), lambda i: (i, 0))],
    )(x, W_l, W_r, b_l.reshape(1, C), b_r.reshape(1, C))

    EB = 8000
    ef = pl.pallas_call(
        _edge_body,
        out_shape=jax.ShapeDtypeStruct((E, C), jnp.float32),
        grid=(E // EB,),
        in_specs=[pl.BlockSpec((EB, DE), lambda i: (i, 0)),
                  pl.BlockSpec((DE, C), lambda i: (0, 0))],
        out_specs=pl.BlockSpec((EB, C), lambda i: (i, 0)),
    )(edge_attr, W_e)

    part, partd = _sc_pass(xl, xr, ef,
                           src.reshape(NW, NCH, K), dst.reshape(NW, NCH, K),
                           att)

    FB = 1024
    out_full = pl.pallas_call(
        _final_body,
        out_shape=jax.ShapeDtypeStruct((NPAD, C), jnp.float32),
        grid=(NPAD // FB,),
        in_specs=[pl.BlockSpec((NC, FB, C), lambda i: (0, i, 0)),
                  pl.BlockSpec((NC, FB, 1), lambda i: (0, i, 0)),
                  pl.BlockSpec((1, C), lambda i: (0, 0))],
        out_specs=pl.BlockSpec((FB, C), lambda i: (i, 0)),
    )(part, partd.reshape(NC, NPAD, 1), bias.reshape(1, C))
    return out_full[:N]


# speedup vs baseline: 1.2325x; 1.0080x over previous
import jax
import jax.numpy as jnp
from jax import lax
from jax.experimental import pallas as pl
from jax.experimental.pallas import tpu as pltpu
from jax.experimental.pallas import tpu_sc as plsc

N = 10000
E = 320000
C = 128
DE = 16
NEG_SLOPE = 0.2

NC = 2
NS = 16
NW = NC * NS
EW = E // NW

K = 40
NCH = EW // K
NG = (K + 15) // 16

NPAD = 10240
RPT = NPAD // NS

_SC_PARAMS = pltpu.CompilerParams(needs_layout_passes=False,
                                  use_tc_tiling_on_sc=False)


def _dense_body(x_ref, wl_ref, wr_ref, bl_ref, br_ref, xl_ref, xr_ref):
    xv = x_ref[...]
    xl_ref[...] = jnp.dot(xv, wl_ref[...],
                          preferred_element_type=jnp.float32) + bl_ref[...]
    xr_ref[...] = jnp.dot(xv, wr_ref[...],
                          preferred_element_type=jnp.float32) + br_ref[...]


def _edge_body(ea_ref, we_ref, ef_ref):
    ef_ref[...] = jnp.dot(ea_ref[...], we_ref[...],
                          preferred_element_type=jnp.float32)


def _final_body(p_ref, pd_ref, bias_ref, out_ref):
    num = p_ref[0] + p_ref[1]
    den = pd_ref[0] + pd_ref[1]
    out_ref[...] = num / (den + 1e-16) + bias_ref[...]


def _sc_body(xl_hbm, xr_hbm, ef_hbm, src3_hbm, dst3_hbm, att_hbm,
             part_hbm, partd_hbm,
             acc_sh, accd_sh,
             src_i, dst_i, sc_i, xl_rows, xr_rows, ef_rows,
             ex_b, accbuf, att_v, zbuf,
             s_ix0, s_ix1, s_xl0, s_xl1, s_xr0, s_xr1, s_ef0, s_ef1,
             s_sc0, s_sc1, s_sd0, s_sd1):
    cid = lax.axis_index("c")
    sid = lax.axis_index("s")
    wid = cid * NS + sid
    zero16 = jnp.zeros((16,), jnp.float32)
    iota = lax.iota(jnp.int32, 16)
    s_ix = [s_ix0, s_ix1]
    s_xl = [s_xl0, s_xl1]
    s_xr = [s_xr0, s_xr1]
    s_ef = [s_ef0, s_ef1]
    s_sc = [s_sc0, s_sc1]
    s_sd = [s_sd0, s_sd1]

    def _z_row(e, _):
        for j in range(C // 16):
            xl_rows[0, e, pl.ds(16 * j, 16)] = zero16
        return 0
    lax.fori_loop(0, K, _z_row, 0)

    def _z1(e, _):
        zbuf[pl.ds(e * 16, 16)] = zero16
        return 0
    lax.fori_loop(0, RPT // 16, _z1, 0)
    row0 = sid * RPT
    for k in range(RPT // K):
        pltpu.sync_copy(xl_rows.at[0], acc_sh.at[pl.ds(row0 + k * K, K)])
    pltpu.sync_copy(zbuf, accd_sh.at[pl.ds(row0, RPT)])
    plsc.subcore_barrier()

    pltpu.sync_copy(att_hbm, att_v)
    att_j = [att_v[pl.ds(16 * j, 16)] for j in range(C // 16)]

    def issue_idx(c, b):
        pltpu.async_copy(src3_hbm.at[pl.ds(wid * EW + c * K, K)],
                         src_i.at[b, pl.ds(0, K)], s_ix[b])
        pltpu.async_copy(dst3_hbm.at[pl.ds(wid * EW + c * K, K)],
                         dst_i.at[b, pl.ds(0, K)], s_ix[b])

    def wait_idx(b):
        pltpu.make_async_copy(src3_hbm.at[pl.ds(0, K)],
                              src_i.at[b, pl.ds(0, K)], s_ix[b]).wait()
        pltpu.make_async_copy(src3_hbm.at[pl.ds(0, K)],
                              dst_i.at[b, pl.ds(0, K)], s_ix[b]).wait()

    def issue_gathers(c, b):
        pltpu.async_copy(xl_hbm.at[src_i.at[b, pl.ds(0, K)]],
                         xl_rows.at[b], s_xl[b])
        pltpu.async_copy(xr_hbm.at[dst_i.at[b, pl.ds(0, K)]],
                         xr_rows.at[b], s_xr[b])
        pltpu.async_copy(ef_hbm.at[pl.ds(wid * EW + c * K, K)],
                         ef_rows.at[b], s_ef[b])

    def wait_gathers(b):
        pltpu.make_async_copy(xl_hbm.at[pl.ds(0, K)],
                              xl_rows.at[b], s_xl[b]).wait()
        pltpu.make_async_copy(xl_hbm.at[pl.ds(0, K)],
                              xr_rows.at[b], s_xr[b]).wait()
        pltpu.make_async_copy(ef_hbm.at[pl.ds(0, K)],
                              ef_rows.at[b], s_ef[b]).wait()

    def drain_scatters(b):
        pltpu.make_async_copy(xl_hbm.at[pl.ds(0, K)],
                              xl_rows.at[b], s_sc[b]).wait()
        pltpu.make_async_copy(partd_hbm.at[0, pl.ds(0, K)],
                              ex_b.at[b, pl.ds(0, K)], s_sd[b]).wait()

    def slot(c, b):
        o = b ^ 1

        @pl.when(c >= 1)
        def _():
            drain_scatters(o)

        @pl.when(c + 1 < NCH)
        def _():
            wait_idx(o)
            issue_gathers(c + 1, o)
        wait_gathers(b)

        for off in (0, 16, K - 16):
            sc_i[b, pl.ds(off, 16)] = dst_i[b, pl.ds(off, 16)]

        @plsc.parallel_loop(0, K, unroll=4)
        def _edge_acc(e):
            acc0 = zero16
            acc1 = zero16
            for j in range(C // 16):
                m = (xl_rows[b, e, pl.ds(16 * j, 16)]
                     + xr_rows[b, e, pl.ds(16 * j, 16)]
                     + ef_rows[b, e, pl.ds(16 * j, 16)])
                m = jnp.maximum(m, NEG_SLOPE * m)
                if j % 2 == 0:
                    acc0 = acc0 + m * att_j[j]
                else:
                    acc1 = acc1 + m * att_j[j]
            accbuf[pl.ds(e * 16, 16)] = acc0 + acc1

        for g in range(NG):
            al = [zero16] * 4
            for col in range(16):
                al[col % 4] = al[col % 4] + plsc.load_gather(
                    accbuf, [iota * 16 + (col + g * 256)])
            ex_b[b, pl.ds(g * 16, 16)] = jnp.exp((al[0] + al[1])
                                                 + (al[2] + al[3]))

        @plsc.parallel_loop(0, K, unroll=4)
        def _edge_scale(e):
            s = plsc.load_gather(ex_b.at[b],
                                 [jnp.zeros((16,), jnp.int32) + e])
            for j in range(C // 16):
                xl_rows[b, e, pl.ds(16 * j, 16)] = (
                    xl_rows[b, e, pl.ds(16 * j, 16)] * s)

        pltpu.async_copy(xl_rows.at[b], acc_sh.at[sc_i.at[b]], s_sc[b],
                         add=True)
        pltpu.async_copy(ex_b.at[b, pl.ds(0, K)], accd_sh.at[sc_i.at[b]],
                         s_sd[b], add=True)

        @pl.when(c + 2 < NCH)
        def _():
            issue_idx(c + 2, b)

    pltpu.sync_copy(src3_hbm.at[pl.ds(wid * EW, K)], src_i.at[0, pl.ds(0, K)])
    pltpu.sync_copy(dst3_hbm.at[pl.ds(wid * EW, K)], dst_i.at[0, pl.ds(0, K)])
    issue_gathers(0, 0)
    issue_idx(1, 1)

    def _pair(p, _):
        slot(2 * p, 0)
        slot(2 * p + 1, 1)
        return 0
    lax.fori_loop(0, NCH // 2, _pair, 0)

    drain_scatters(1)

    plsc.subcore_barrier()
    pltpu.sync_copy(acc_sh.at[pl.ds(row0, RPT)],
                    part_hbm.at[cid, pl.ds(row0, RPT)])
    pltpu.sync_copy(accd_sh.at[pl.ds(row0, RPT)],
                    partd_hbm.at[cid, pl.ds(row0, RPT)])


def _sc_pass(xl, xr, ef, src3, dst3, att):
    mesh = plsc.VectorSubcoreMesh(core_axis_name="c", subcore_axis_name="s",
                                  num_cores=NC, num_subcores=NS)
    f = pl.kernel(
        _sc_body,
        out_type=[jax.ShapeDtypeStruct((NC, NPAD, C), jnp.float32),
                  jax.ShapeDtypeStruct((NC, NPAD), jnp.float32)],
        mesh=mesh,
        scratch_types=[
            pltpu.VMEM_SHARED((NPAD, C), jnp.float32),
            pltpu.VMEM_SHARED((NPAD,), jnp.float32),
            pltpu.VMEM((2, 48), jnp.int32),
            pltpu.VMEM((2, 48), jnp.int32),
            pltpu.VMEM((2, K), jnp.int32),
            pltpu.VMEM((2, K, C), jnp.float32),
            pltpu.VMEM((2, K, C), jnp.float32),
            pltpu.VMEM((2, K, C), jnp.float32),
            pltpu.VMEM((2, NG * 16), jnp.float32),
            pltpu.VMEM((NG * 256, ), jnp.float32),
            pltpu.VMEM((C,), jnp.float32),
            pltpu.VMEM((RPT,), jnp.float32),
        ] + [pltpu.SemaphoreType.DMA] * 12,
        compiler_params=_SC_PARAMS,
    )
    return f(xl, xr, ef, src3, dst3, att)


def kernel(x, edge_index, edge_attr, W_l, b_l, W_r, b_r, W_e, att, bias):
    src = edge_index[0]
    dst = edge_index[1]

    xl, xr = pl.pallas_call(
        _dense_body,
        out_shape=[jax.ShapeDtypeStruct((N, C), jnp.float32),
                   jax.ShapeDtypeStruct((N, C), jnp.float32)],
        grid=(5,),
        in_specs=[pl.BlockSpec((N // 5, C), lambda i: (i, 0)),
                  pl.BlockSpec((C, C), lambda i: (0, 0)),
                  pl.BlockSpec((C, C), lambda i: (0, 0)),
                  pl.BlockSpec((1, C), lambda i: (0, 0)),
                  pl.BlockSpec((1, C), lambda i: (0, 0))],
        out_specs=[pl.BlockSpec((N // 5, C), lambda i: (i, 0)),
                   pl.BlockSpec((N // 5, C), lambda i: (i, 0))],
    )(x, W_l, W_r, b_l.reshape(1, C), b_r.reshape(1, C))

    EB = 8000
    ef = pl.pallas_call(
        _edge_body,
        out_shape=jax.ShapeDtypeStruct((E, C), jnp.float32),
        grid=(E // EB,),
        in_specs=[pl.BlockSpec((EB, DE), lambda i: (i, 0)),
                  pl.BlockSpec((DE, C), lambda i: (0, 0))],
        out_specs=pl.BlockSpec((EB, C), lambda i: (i, 0)),
    )(edge_attr, W_e)

    part, partd = _sc_pass(xl, xr, ef, src, dst, att)

    FB = 1000
    out = pl.pallas_call(
        _final_body,
        out_shape=jax.ShapeDtypeStruct((N, C), jnp.float32),
        grid=(N // FB,),
        in_specs=[pl.BlockSpec((NC, FB, C), lambda i: (0, i, 0)),
                  pl.BlockSpec((NC, FB, 1), lambda i: (0, i, 0)),
                  pl.BlockSpec((1, C), lambda i: (0, 0))],
        out_specs=pl.BlockSpec((FB, C), lambda i: (i, 0)),
    )(part, partd.reshape(NC, NPAD, 1), bias.reshape(1, C))
    return out


# speedup vs baseline: 1.4913x; 1.2100x over previous
import jax
import jax.numpy as jnp
from jax import lax
from jax.experimental import pallas as pl
from jax.experimental.pallas import tpu as pltpu
from jax.experimental.pallas import tpu_sc as plsc

N = 10000
E = 320000
C = 128
DE = 16
NEG_SLOPE = 0.2

NC = 2
NS = 16
NW = NC * NS
EW = E // NW

K = 40
NCH = EW // K
NG = (K + 15) // 16

NPAD = 10240
RPT = NPAD // NS

_SC_PARAMS = pltpu.CompilerParams(needs_layout_passes=False,
                                  use_tc_tiling_on_sc=False)


def _dense_body(x_ref, wl_ref, wr_ref, bl_ref, br_ref, xl_ref, xr_ref):
    xv = x_ref[...]
    xl_ref[...] = jnp.dot(xv, wl_ref[...],
                          preferred_element_type=jnp.float32) + bl_ref[...]
    xr_ref[...] = jnp.dot(xv, wr_ref[...],
                          preferred_element_type=jnp.float32) + br_ref[...]


def _edge_body(eat_ref, we_ref, ef_ref):
    ef_ref[...] = lax.dot_general(
        eat_ref[...], we_ref[...],
        dimension_numbers=(((0,), (0,)), ((), ())),
        preferred_element_type=jnp.float32)


def _final_body(p_ref, pd_ref, bias_ref, out_ref):
    num = p_ref[0] + p_ref[1]
    den = pd_ref[0] + pd_ref[1]
    out_ref[...] = num / (den + 1e-16) + bias_ref[...]


def _sc_body(xl_hbm, xr_hbm, ef_hbm, src3_hbm, dst3_hbm, att_hbm,
             part_hbm, partd_hbm,
             acc_sh, accd_sh,
             src_i, dst_i, sc_i, xl_rows, xr_rows, ef_rows,
             ex_b, accbuf, att_v, zbuf,
             s_ix0, s_ix1, s_xl0, s_xl1, s_xr0, s_xr1, s_ef0, s_ef1,
             s_sc0, s_sc1, s_sd0, s_sd1):
    cid = lax.axis_index("c")
    sid = lax.axis_index("s")
    wid = cid * NS + sid
    zero16 = jnp.zeros((16,), jnp.float32)
    iota = lax.iota(jnp.int32, 16)
    s_ix = [s_ix0, s_ix1]
    s_xl = [s_xl0, s_xl1]
    s_xr = [s_xr0, s_xr1]
    s_ef = [s_ef0, s_ef1]
    s_sc = [s_sc0, s_sc1]
    s_sd = [s_sd0, s_sd1]

    def _z_row(e, _):
        for j in range(C // 16):
            xl_rows[0, e, pl.ds(16 * j, 16)] = zero16
        return 0
    lax.fori_loop(0, K, _z_row, 0)

    def _z1(e, _):
        zbuf[pl.ds(e * 16, 16)] = zero16
        return 0
    lax.fori_loop(0, RPT // 16, _z1, 0)
    row0 = sid * RPT
    for k in range(RPT // K):
        pltpu.sync_copy(xl_rows.at[0], acc_sh.at[pl.ds(row0 + k * K, K)])
    pltpu.sync_copy(zbuf, accd_sh.at[pl.ds(row0, RPT)])
    plsc.subcore_barrier()

    pltpu.sync_copy(att_hbm, att_v)
    att_j = [att_v[pl.ds(16 * j, 16)] for j in range(C // 16)]

    def issue_idx(c, b):
        pltpu.async_copy(src3_hbm.at[pl.ds(wid * EW + c * K, K)],
                         src_i.at[b, pl.ds(0, K)], s_ix[b])
        pltpu.async_copy(dst3_hbm.at[pl.ds(wid * EW + c * K, K)],
                         dst_i.at[b, pl.ds(0, K)], s_ix[b])

    def wait_idx(b):
        pltpu.make_async_copy(src3_hbm.at[pl.ds(0, K)],
                              src_i.at[b, pl.ds(0, K)], s_ix[b]).wait()
        pltpu.make_async_copy(src3_hbm.at[pl.ds(0, K)],
                              dst_i.at[b, pl.ds(0, K)], s_ix[b]).wait()

    def issue_gathers(c, b):
        pltpu.async_copy(xl_hbm.at[src_i.at[b, pl.ds(0, K)]],
                         xl_rows.at[b], s_xl[b])
        pltpu.async_copy(xr_hbm.at[dst_i.at[b, pl.ds(0, K)]],
                         xr_rows.at[b], s_xr[b])
        pltpu.async_copy(ef_hbm.at[pl.ds(wid * EW + c * K, K)],
                         ef_rows.at[b], s_ef[b])

    def wait_gathers(b):
        pltpu.make_async_copy(xl_hbm.at[pl.ds(0, K)],
                              xl_rows.at[b], s_xl[b]).wait()
        pltpu.make_async_copy(xl_hbm.at[pl.ds(0, K)],
                              xr_rows.at[b], s_xr[b]).wait()
        pltpu.make_async_copy(ef_hbm.at[pl.ds(0, K)],
                              ef_rows.at[b], s_ef[b]).wait()

    def drain_scatters(b):
        pltpu.make_async_copy(xl_hbm.at[pl.ds(0, K)],
                              xl_rows.at[b], s_sc[b]).wait()
        pltpu.make_async_copy(partd_hbm.at[0, pl.ds(0, K)],
                              ex_b.at[b, pl.ds(0, K)], s_sd[b]).wait()

    def slot(c, b):
        o = b ^ 1

        @pl.when(c >= 1)
        def _():
            drain_scatters(o)

        @pl.when(c + 1 < NCH)
        def _():
            wait_idx(o)
            issue_gathers(c + 1, o)
        wait_gathers(b)

        for off in (0, 16, K - 16):
            sc_i[b, pl.ds(off, 16)] = dst_i[b, pl.ds(off, 16)]

        @plsc.parallel_loop(0, K, unroll=4)
        def _edge_acc(e):
            acc0 = zero16
            acc1 = zero16
            for j in range(C // 16):
                m = (xl_rows[b, e, pl.ds(16 * j, 16)]
                     + xr_rows[b, e, pl.ds(16 * j, 16)]
                     + ef_rows[b, e, pl.ds(16 * j, 16)])
                m = jnp.maximum(m, NEG_SLOPE * m)
                if j % 2 == 0:
                    acc0 = acc0 + m * att_j[j]
                else:
                    acc1 = acc1 + m * att_j[j]
            accbuf[pl.ds(e * 16, 16)] = acc0 + acc1

        for g in range(NG):
            al = [zero16] * 4
            for col in range(16):
                al[col % 4] = al[col % 4] + plsc.load_gather(
                    accbuf, [iota * 16 + (col + g * 256)])
            ex_b[b, pl.ds(g * 16, 16)] = jnp.exp((al[0] + al[1])
                                                 + (al[2] + al[3]))

        @plsc.parallel_loop(0, K, unroll=4)
        def _edge_scale(e):
            s = plsc.load_gather(ex_b.at[b],
                                 [jnp.zeros((16,), jnp.int32) + e])
            for j in range(C // 16):
                xl_rows[b, e, pl.ds(16 * j, 16)] = (
                    xl_rows[b, e, pl.ds(16 * j, 16)] * s)

        pltpu.async_copy(xl_rows.at[b], acc_sh.at[sc_i.at[b]], s_sc[b],
                         add=True)
        pltpu.async_copy(ex_b.at[b, pl.ds(0, K)], accd_sh.at[sc_i.at[b]],
                         s_sd[b], add=True)

        @pl.when(c + 2 < NCH)
        def _():
            issue_idx(c + 2, b)

    pltpu.sync_copy(src3_hbm.at[pl.ds(wid * EW, K)], src_i.at[0, pl.ds(0, K)])
    pltpu.sync_copy(dst3_hbm.at[pl.ds(wid * EW, K)], dst_i.at[0, pl.ds(0, K)])
    issue_gathers(0, 0)
    issue_idx(1, 1)

    def _pair(p, _):
        slot(2 * p, 0)
        slot(2 * p + 1, 1)
        return 0
    lax.fori_loop(0, NCH // 2, _pair, 0)

    drain_scatters(1)

    plsc.subcore_barrier()
    pltpu.sync_copy(acc_sh.at[pl.ds(row0, RPT)],
                    part_hbm.at[cid, pl.ds(row0, RPT)])
    pltpu.sync_copy(accd_sh.at[pl.ds(row0, RPT)],
                    partd_hbm.at[cid, pl.ds(row0, RPT)])


def _sc_pass(xl, xr, ef, src3, dst3, att):
    mesh = plsc.VectorSubcoreMesh(core_axis_name="c", subcore_axis_name="s",
                                  num_cores=NC, num_subcores=NS)
    f = pl.kernel(
        _sc_body,
        out_type=[jax.ShapeDtypeStruct((NC, NPAD, C), jnp.float32),
                  jax.ShapeDtypeStruct((NC, NPAD), jnp.float32)],
        mesh=mesh,
        scratch_types=[
            pltpu.VMEM_SHARED((NPAD, C), jnp.float32),
            pltpu.VMEM_SHARED((NPAD,), jnp.float32),
            pltpu.VMEM((2, 48), jnp.int32),
            pltpu.VMEM((2, 48), jnp.int32),
            pltpu.VMEM((2, K), jnp.int32),
            pltpu.VMEM((2, K, C), jnp.float32),
            pltpu.VMEM((2, K, C), jnp.float32),
            pltpu.VMEM((2, K, C), jnp.float32),
            pltpu.VMEM((2, NG * 16), jnp.float32),
            pltpu.VMEM((NG * 256, ), jnp.float32),
            pltpu.VMEM((C,), jnp.float32),
            pltpu.VMEM((RPT,), jnp.float32),
        ] + [pltpu.SemaphoreType.DMA] * 12,
        compiler_params=_SC_PARAMS,
    )
    return f(xl, xr, ef, src3, dst3, att)


def kernel(x, edge_index, edge_attr, W_l, b_l, W_r, b_r, W_e, att, bias):
    src = edge_index[0]
    dst = edge_index[1]

    xl, xr = pl.pallas_call(
        _dense_body,
        out_shape=[jax.ShapeDtypeStruct((N, C), jnp.float32),
                   jax.ShapeDtypeStruct((N, C), jnp.float32)],
        grid=(5,),
        in_specs=[pl.BlockSpec((N // 5, C), lambda i: (i, 0)),
                  pl.BlockSpec((C, C), lambda i: (0, 0)),
                  pl.BlockSpec((C, C), lambda i: (0, 0)),
                  pl.BlockSpec((1, C), lambda i: (0, 0)),
                  pl.BlockSpec((1, C), lambda i: (0, 0))],
        out_specs=[pl.BlockSpec((N // 5, C), lambda i: (i, 0)),
                   pl.BlockSpec((N // 5, C), lambda i: (i, 0))],
    )(x, W_l, W_r, b_l.reshape(1, C), b_r.reshape(1, C))

    EB = 6400
    ef = pl.pallas_call(
        _edge_body,
        out_shape=jax.ShapeDtypeStruct((E, C), jnp.float32),
        grid=(E // EB,),
        in_specs=[pl.BlockSpec((DE, EB), lambda i: (0, i)),
                  pl.BlockSpec((DE, C), lambda i: (0, 0))],
        out_specs=pl.BlockSpec((EB, C), lambda i: (i, 0)),
    )(edge_attr.T, W_e)

    part, partd = _sc_pass(xl, xr, ef, src, dst, att)

    FB = 1000
    out = pl.pallas_call(
        _final_body,
        out_shape=jax.ShapeDtypeStruct((N, C), jnp.float32),
        grid=(N // FB,),
        in_specs=[pl.BlockSpec((NC, FB, C), lambda i: (0, i, 0)),
                  pl.BlockSpec((NC, FB, 1), lambda i: (0, i, 0)),
                  pl.BlockSpec((1, C), lambda i: (0, 0))],
        out_specs=pl.BlockSpec((FB, C), lambda i: (i, 0)),
    )(part, partd.reshape(NC, NPAD, 1), bias.reshape(1, C))
    return out
